# Initial kernel scaffold; baseline (speedup 1.0000x reference)
import jax
import jax.numpy as jnp
from jax.experimental import pallas as pl


def kernel(drug, edge_index, batch, W_gcn, b_gcn, ln_w, ln_b, mu, sigma):
    raise NotImplementedError("write your pallas kernel here")


# speedup vs baseline: 20.5066x; 20.5066x over previous
import functools

import jax
import jax.numpy as jnp
from jax import lax
from jax.experimental import pallas as pl
from jax.experimental.pallas import tpu as pltpu
from jax.experimental.pallas import tpu_sc as plsc

N = 10000
E = 320000
G = 64
D = 128
DH = D // 2
NP = 10240
NC = 2
NS = 16
EW = E // (NC * NS)
K = 80
KD = 80
NCH = EW // KD
EW2 = E // NS
NCH2 = EW2 // K
RPT = NP // NS
BLK = 1280
GRID = NP // BLK


def _sc_mesh():
    return plsc.VectorSubcoreMesh(core_axis_name="c", subcore_axis_name="s")


def _tc_matmul(x, w):
    def body(xr, wr, outr):
        outr[...] = jnp.dot(xr[...], wr[...], preferred_element_type=jnp.float32)

    return pl.pallas_call(
        body,
        grid=(GRID,),
        in_specs=[
            pl.BlockSpec((BLK, D), lambda i: (i, 0)),
            pl.BlockSpec((D, D), lambda i: (0, 0)),
        ],
        out_specs=pl.BlockSpec((BLK, D), lambda i: (i, 0)),
        out_shape=jax.ShapeDtypeStruct((NP, D), jnp.float32),
    )(x, w)


def _sc_degree(dst_r):
    @functools.partial(
        pl.kernel,
        out_type=jax.ShapeDtypeStruct((NC, NP), jnp.float32),
        mesh=_sc_mesh(),
        scratch_types=[
            pltpu.VMEM((NCH, KD), jnp.int32),
            pltpu.VMEM((KD,), jnp.float32),
            pltpu.VMEM((RPT,), jnp.float32),
            pltpu.VMEM_SHARED((NP,), jnp.float32),
        ],
    )
    def k(dst_hbm, out_hbm, idx2d, ones, zbuf, acc):
        c = lax.axis_index("c")
        s = lax.axis_index("s")
        zero16 = jnp.zeros((16,), jnp.float32)
        one16 = jnp.ones((16,), jnp.float32)

        def zb(i, carry):
            zbuf[pl.ds(i * 16, 16)] = zero16
            return carry

        lax.fori_loop(0, RPT // 16, zb, None)
        for i in range(KD // 16):
            ones[pl.ds(i * 16, 16)] = one16
        pltpu.sync_copy(zbuf, acc.at[pl.ds(s * RPT, RPT)])
        pltpu.sync_copy(dst_hbm.at[c, s], idx2d)
        plsc.subcore_barrier()

        def body(j, carry):
            pltpu.sync_copy(ones, acc.at[idx2d.at[j]], add=True)
            return carry

        lax.fori_loop(0, NCH, body, None)
        plsc.subcore_barrier()
        pltpu.sync_copy(acc.at[pl.ds(s * RPT, RPT)],
                        out_hbm.at[c, pl.ds(s * RPT, RPT)])

    return k(dst_r)


def _tc_scale(degp, xw):
    def body(dr, xr, yr, dvr):
        d = dr[0] + dr[1] + 1.0
        dv = lax.rsqrt(d)
        dvr[...] = dv
        yr[...] = xr[...] * dv

    return pl.pallas_call(
        body,
        grid=(GRID,),
        in_specs=[
            pl.BlockSpec((NC, BLK, 1), lambda i: (0, i, 0)),
            pl.BlockSpec((BLK, D), lambda i: (i, 0)),
        ],
        out_specs=[
            pl.BlockSpec((BLK, D), lambda i: (i, 0)),
            pl.BlockSpec((BLK, 1), lambda i: (i, 0)),
        ],
        out_shape=[
            jax.ShapeDtypeStruct((NP, D), jnp.float32),
            jax.ShapeDtypeStruct((NP, 1), jnp.float32),
        ],
    )(degp, xw)


HALF = NP // 2
DUMP = 128
AROWS = HALF + DUMP
ZPT = AROWS // NS
OPT = HALF // NS
NG = EW2 // 16
CAP = ((EW2 + 2 * K - 1) // (2 * K) + 1) * (2 * K)


CH = 125
NCK = NG // CH


def _sc_compact(src_r, dst_r):
    @functools.partial(
        pl.kernel,
        out_type=[
            jax.ShapeDtypeStruct((NC, NS, CAP), jnp.int32),
            jax.ShapeDtypeStruct((NC, NS, CAP), jnp.int32),
            jax.ShapeDtypeStruct((NC, NS, 16), jnp.int32),
        ],
        mesh=_sc_mesh(),
        compiler_params=pltpu.CompilerParams(needs_layout_passes=False),
        scratch_types=[
            pltpu.VMEM((CH, 16), jnp.int32),
            pltpu.VMEM((CH, 16), jnp.int32),
            pltpu.VMEM((CH, 16), jnp.int32),
            pltpu.VMEM((CH, 16), jnp.int32),
            pltpu.VMEM((CAP,), jnp.int32),
            pltpu.VMEM((CAP,), jnp.int32),
            pltpu.VMEM((16,), jnp.int32),
            pltpu.SemaphoreType.DMA,
            pltpu.SemaphoreType.DMA,
        ],
    )
    def k(src_hbm, dst_hbm, cs_hbm, cd_hbm, cnt_hbm,
          sb0, sb1, db0, db1, csrc, cdst, cbuf, sma, smb):
        c = lax.axis_index("c")
        s = lax.axis_index("s")
        lane = lax.iota(jnp.int32, 16)
        off = c * HALF

        sbufs = [sb0, sb1]
        dbufs = [db0, db1]
        cps = pltpu.async_copy(src_hbm.at[s, 0], sb0, sma)
        cpd = pltpu.async_copy(dst_hbm.at[s, 0], db0, smb)
        cnt = jnp.int32(0)
        for tt in range(NCK):
            sb = sbufs[tt % 2]
            db = dbufs[tt % 2]
            cps.wait()
            cpd.wait()
            if tt + 1 < NCK:
                cps = pltpu.async_copy(
                    src_hbm.at[s, tt + 1], sbufs[(tt + 1) % 2], sma)
                cpd = pltpu.async_copy(
                    dst_hbm.at[s, tt + 1], dbufs[(tt + 1) % 2], smb)

            def comp(g, cnt2):
                sv = sb[g]
                dv = db[g]
                dloc = dv - off
                m = (dloc >= 0) & (dloc < HALF)
                mi = m.astype(jnp.int32)
                pos = cnt2 + plsc.cumsum(mi) - 1
                plsc.store_scatter(csrc, [pos], sv, mask=m)
                plsc.store_scatter(cdst, [pos], dloc, mask=m)
                return cnt2 + jnp.sum(mi)

            cnt = lax.fori_loop(0, CH, comp, cnt)

        target = ((cnt + 2 * K - 1) // (2 * K)) * (2 * K)
        dump_dst = HALF + (lane * 8 + s % 8)
        for _ in range(2 * K // 16):
            m = lane < (target - cnt)
            pos = cnt + lane
            plsc.store_scatter(csrc, [pos], lane, mask=m)
            plsc.store_scatter(cdst, [pos], dump_dst, mask=m)
            cnt = cnt + jnp.sum(m.astype(jnp.int32))

        cbuf[pl.ds(0, 16)] = jnp.zeros((16,), jnp.int32) + target
        pltpu.sync_copy(csrc, cs_hbm.at[c, s])
        pltpu.sync_copy(cdst, cd_hbm.at[c, s])
        pltpu.sync_copy(cbuf, cnt_hbm.at[c, s])

    return k(src_r, dst_r)


def _sc_edge_sum(cs, cd, cnt, y):
    @functools.partial(
        pl.kernel,
        out_type=jax.ShapeDtypeStruct((NP, D), jnp.float32),
        mesh=_sc_mesh(),
        scratch_types=[
            pltpu.VMEM((CAP,), jnp.int32),
            pltpu.VMEM((CAP,), jnp.int32),
            pltpu.VMEM((16,), jnp.int32),
            pltpu.VMEM((K,), jnp.int32),
            pltpu.VMEM((K,), jnp.int32),
            pltpu.VMEM((K,), jnp.int32),
            pltpu.VMEM((K,), jnp.int32),
            pltpu.VMEM((K, D), jnp.float32),
            pltpu.VMEM((K, D), jnp.float32),
            pltpu.VMEM((K, D), jnp.float32),
            pltpu.VMEM((K, D), jnp.float32),
            pltpu.VMEM_SHARED((AROWS, D), jnp.float32),
            pltpu.SemaphoreType.DMA,
            pltpu.SemaphoreType.DMA,
            pltpu.SemaphoreType.DMA,
            pltpu.SemaphoreType.DMA,
        ],
    )
    def k(cs_hbm, cd_hbm, cnt_hbm, y_hbm, out_hbm,
          csrc, cdst, cbuf, sd0, sd1, sd2, sd3,
          rows0, rows1, rows2, rows3, acc, sem0, sem1, sem2, sem3):
        c = lax.axis_index("c")
        s = lax.axis_index("s")
        zero16 = jnp.zeros((16,), jnp.float32)

        def zrow(a, carry):
            def zlane(l, carry2):
                rows0[a, pl.ds(l * 16, 16)] = zero16
                return carry2
            return lax.fori_loop(0, D // 16, zlane, carry)

        lax.fori_loop(0, K, zrow, None)
        for t in range(ZPT // K):
            pltpu.sync_copy(rows0, acc.at[pl.ds(s * ZPT + t * K, K)])
        rem = ZPT % K
        if rem:
            pltpu.sync_copy(rows0.at[pl.ds(0, rem)],
                            acc.at[pl.ds(s * ZPT + (ZPT // K) * K, rem)])
        pltpu.sync_copy(cs_hbm.at[c, s], csrc)
        pltpu.sync_copy(cd_hbm.at[c, s], cdst)
        pltpu.sync_copy(cnt_hbm.at[c, s], cbuf)
        plsc.subcore_barrier()

        target = cbuf[pl.ds(0, 16)][0]

        rows = [rows0, rows1, rows2, rows3]
        sds = [sd0, sd1, sd2, sd3]
        sems = [sem0, sem1, sem2, sem3]

        def quad(q, carry):
            jb = q * 4 * K
            cps = []
            for u in range(4):
                cps.append(pltpu.async_copy(
                    y_hbm.at[csrc.at[pl.ds(jb + u * K, K)]], rows[u], sems[u]))
            for u in range(4):
                for i in range(K // 16):
                    sds[u][pl.ds(i * 16, 16)] = cdst[pl.ds(jb + u * K + i * 16, 16)]
            for u in range(4):
                cps[u].wait()
                pltpu.sync_copy(rows[u], acc.at[sds[u]], add=True)
            return carry

        nq = target // (4 * K)
        lax.fori_loop(0, nq, quad, None)

        @pl.when(target - nq * 4 * K >= 2 * K)
        def _():
            jb = nq * 4 * K
            cp0 = pltpu.async_copy(y_hbm.at[csrc.at[pl.ds(jb, K)]], rows0, sem0)
            cp1 = pltpu.async_copy(y_hbm.at[csrc.at[pl.ds(jb + K, K)]], rows1, sem1)
            for i in range(K // 16):
                sd0[pl.ds(i * 16, 16)] = cdst[pl.ds(jb + i * 16, 16)]
                sd1[pl.ds(i * 16, 16)] = cdst[pl.ds(jb + K + i * 16, 16)]
            cp0.wait()
            pltpu.sync_copy(rows0, acc.at[sd0], add=True)
            cp1.wait()
            pltpu.sync_copy(rows1, acc.at[sd1], add=True)

        plsc.subcore_barrier()
        pltpu.sync_copy(acc.at[pl.ds(s * OPT, OPT)],
                        out_hbm.at[pl.ds(c * HALF + s * OPT, OPT)])

    return k(cs, cd, cnt, y)


def _tc_combine_stats(accp, y, dinv, b, batch_row):
    def body(ar, yr, dvr, br, btr, gr, st):
        i = pl.program_id(0)
        g = dvr[...] * (ar[...] + yr[...]) + br[...]
        gr[...] = g
        gids = lax.broadcasted_iota(jnp.int32, (G, 1), 0)
        oh = (btr[...] == gids).astype(jnp.float32)
        s1 = jnp.dot(oh, g, preferred_element_type=jnp.float32)
        s2 = jnp.dot(oh, g * g, preferred_element_type=jnp.float32)
        cnt = jnp.sum(oh, axis=1, keepdims=True)
        s3 = jnp.broadcast_to(cnt, (G, D))
        upd = jnp.stack([s1, s2, s3])

        @pl.when(i == 0)
        def _():
            st[...] = jnp.zeros_like(st)

        st[...] += upd

    return pl.pallas_call(
        body,
        grid=(GRID,),
        in_specs=[
            pl.BlockSpec((BLK, D), lambda i: (i, 0)),
            pl.BlockSpec((BLK, D), lambda i: (i, 0)),
            pl.BlockSpec((BLK, 1), lambda i: (i, 0)),
            pl.BlockSpec((1, D), lambda i: (0, 0)),
            pl.BlockSpec((1, BLK), lambda i: (0, i)),
        ],
        out_specs=[
            pl.BlockSpec((BLK, D), lambda i: (i, 0)),
            pl.BlockSpec((3, G, D), lambda i: (0, 0, 0)),
        ],
        out_shape=[
            jax.ShapeDtypeStruct((NP, D), jnp.float32),
            jax.ShapeDtypeStruct((3, G, D), jnp.float32),
        ],
    )(accp, y, dinv, b, batch_row)


def _tc_norm_fuzzy(g, batch_col, stats, ln_w, ln_b, mu2, sg2):
    SUB = 128

    def body(gr, bcr, st, lwr, lbr, mur, sgr, outr, zs, isr):
        tot1 = jnp.sum(st[0], axis=1, keepdims=True)
        tot2 = jnp.sum(st[1], axis=1, keepdims=True)
        cnt = jnp.maximum(st[2, :, 0:1] * D, 1.0)
        mean = tot1 / cnt
        var = tot2 / cnt - mean * mean
        inv = lax.rsqrt(var + 1e-5)
        gids = lax.broadcasted_iota(jnp.int32, (1, G), 1)
        oh = (bcr[...] == gids).astype(jnp.float32)
        mrow = jnp.dot(oh, mean, preferred_element_type=jnp.float32)
        irow = jnp.dot(oh, inv, preferred_element_type=jnp.float32)
        zs[...] = (gr[...] - mrow) * irow * lwr[...] + lbr[...]
        sg = sgr[...]
        isr[...] = -1.0 / (sg * sg + 1e-4)

        def sub(t, carry):
            zsub = zs[pl.ds(t * SUB, SUB), :]

            def kk(k4, a):
                for u in range(10):
                    m = mur[pl.ds(k4 * 10 + u, 1), :]
                    ii = isr[pl.ds(k4 * 10 + u, 1), :]
                    d = zsub - m
                    a = a + jnp.exp((d * d) * ii)
                return a

            a = lax.fori_loop(0, 10, kk, jnp.zeros((SUB, D), jnp.float32))
            outr[pl.ds(t * SUB, SUB), :] = (a * (1.0 / 20.0)) * gr[pl.ds(t * SUB, SUB), :]
            return carry

        lax.fori_loop(0, BLK // SUB, sub, None)

    return pl.pallas_call(
        body,
        grid=(GRID,),
        in_specs=[
            pl.BlockSpec((BLK, D), lambda i: (i, 0)),
            pl.BlockSpec((BLK, 1), lambda i: (i, 0)),
            pl.BlockSpec((3, G, D), lambda i: (0, 0, 0)),
            pl.BlockSpec((1, D), lambda i: (0, 0)),
            pl.BlockSpec((1, D), lambda i: (0, 0)),
            pl.BlockSpec((100, D), lambda i: (0, 0)),
            pl.BlockSpec((100, D), lambda i: (0, 0)),
        ],
        out_specs=pl.BlockSpec((BLK, D), lambda i: (i, 0)),
        out_shape=jax.ShapeDtypeStruct((NP, D), jnp.float32),
        scratch_shapes=[
            pltpu.VMEM((BLK, D), jnp.float32),
            pltpu.VMEM((100, D), jnp.float32),
        ],
    )(g, batch_col, stats, ln_w, ln_b, mu2, sg2)


def kernel(drug, edge_index, batch, W_gcn, b_gcn, ln_w, ln_b, mu, sigma):
    drug_p = jnp.pad(drug, ((0, NP - N), (0, 0)))
    dst_deg = edge_index[1].reshape(NC, NS, NCH, KD)
    src_r = edge_index[0].reshape(NS, NCK, CH, 16)
    dst_r = edge_index[1].reshape(NS, NCK, CH, 16)
    batch_p = jnp.pad(batch, (0, NP - N), constant_values=100)

    xw = _tc_matmul(drug_p, W_gcn)
    degp = _sc_degree(dst_deg)
    cs, cd, cnt = _sc_compact(src_r, dst_r)
    y, dinv = _tc_scale(degp.reshape(NC, NP, 1), xw)
    acc = _sc_edge_sum(cs, cd, cnt, y)
    g, stats = _tc_combine_stats(acc, y, dinv, b_gcn.reshape(1, D),
                                 batch_p.reshape(1, NP))
    out = _tc_norm_fuzzy(g, batch_p.reshape(NP, 1), stats,
                         ln_w.reshape(1, D), ln_b.reshape(1, D),
                         mu.reshape(5 * 20, D), sigma.reshape(5 * 20, D))
    return out[:N]
